# baseline (speedup 1.0000x reference)
import functools
import math

import jax
import jax.numpy as jnp
from jax.experimental import pallas as pl
from jax.experimental.pallas import tpu as pltpu

_SMOOTHING = 0.1
_CONF = 1.0 - _SMOOTHING
_RB = 512
_CB = 4096
_LANE = 128
_NSPLIT = 4
_SRB = _RB // _NSPLIT


def _loss_body(V, eps, c_row, gj, t_ref, *rest):
    x_refs = rest[:_NSPLIT]
    out_ref = rest[_NSPLIT]
    racc_ref = rest[_NSPLIT + 1]
    i = pl.program_id(0)
    j = pl.program_id(1)
    nsl = _CB // _LANE
    lane = jax.lax.broadcasted_iota(jnp.int32, (8, _LANE), 1)
    whit = _CONF / eps

    @pl.when((i == 0) & (j == 0))
    def _init():
        out_ref[0, 0] = 0.0

    def fold_into(xr, q, t, first, mask_tail):
        block = xr[...]
        if mask_tail:
            cols = (gj - 1) * _CB + jax.lax.broadcasted_iota(
                jnp.int32, (1, _CB), 1)
            block = jnp.where(cols >= V, 0.0, block)
        for r in range(0, _SRB, 8):
            row = q * _SRB + r
            ts = t[row:row + 8, :]
            t_tile = (ts >> 7) - j * nsl
            lane_hit = (ts & 127) == lane
            any_hit = jnp.any((t_tile >= 0) & (t_tile < nsl))

            @pl.when(any_hit)
            def _with_onehot(r=r, row=row, t_tile=t_tile, lane_hit=lane_hit):
                acc = None
                for k in range(nsl):
                    sl = block[r:r + 8, k * _LANE:(k + 1) * _LANE]
                    w = jnp.where(lane_hit & (t_tile == k), whit, 1.0)
                    acc = sl * w if acc is None else acc + sl * w
                if first:
                    racc_ref[row:row + 8, :] = acc
                else:
                    racc_ref[row:row + 8, :] += acc

            @pl.when(jnp.logical_not(any_hit))
            def _plain(r=r, row=row):
                acc = None
                for k in range(nsl):
                    sl = block[r:r + 8, k * _LANE:(k + 1) * _LANE]
                    acc = sl if acc is None else acc + sl
                if first:
                    racc_ref[row:row + 8, :] = acc
                else:
                    racc_ref[row:row + 8, :] += acc

    @pl.when(j == 0)
    def _first():
        t = t_ref[...]
        s = 0.0
        for q, xr in enumerate(x_refs):
            fold_into(xr, q, t, True, gj == 1)
            x0 = xr[0:_SRB, 0:1]
            tq = t[q * _SRB:(q + 1) * _SRB, :]
            per_row = c_row + eps * x0
            s = s + jnp.sum(jnp.where(tq == 0, 0.0, per_row))
        out_ref[0, 0] += s

    if gj > 1:
        @pl.when((j > 0) & (j < gj - 1))
        def _mid():
            t = t_ref[...]
            for q, xr in enumerate(x_refs):
                fold_into(xr, q, t, False, False)

        @pl.when(j == gj - 1)
        def _tail():
            t = t_ref[...]
            for q, xr in enumerate(x_refs):
                fold_into(xr, q, t, False, True)

    @pl.when(j == gj - 1)
    def _last():
        rowsum = jnp.sum(racc_ref[...], axis=1, keepdims=True)
        t = t_ref[...]
        out_ref[0, 0] += -eps * jnp.sum(jnp.where(t == 0, 0.0, rowsum))


def kernel(x, target):
    n, V = x.shape
    eps = _SMOOTHING / (V - 2)
    c_row = _SMOOTHING * math.log(eps) + _CONF * math.log(_CONF)
    t2 = target.astype(jnp.int32).reshape(n, 1)
    gi = n // _RB
    gj = pl.cdiv(V, _CB)

    out = pl.pallas_call(
        functools.partial(_loss_body, V, eps, c_row, gj),
        grid=(gi, gj),
        in_specs=[
            pl.BlockSpec((_RB, 1), lambda i, j: (i, 0)),
        ] + [
            pl.BlockSpec((_SRB, _CB),
                         lambda i, j, q=q: (_NSPLIT * i + q, j))
            for q in range(_NSPLIT)
        ],
        out_specs=pl.BlockSpec((1, 1), lambda i, j: (0, 0),
                               memory_space=pltpu.SMEM),
        out_shape=jax.ShapeDtypeStruct((1, 1), jnp.float32),
        scratch_shapes=[pltpu.VMEM((_RB, _LANE), jnp.float32)],
        compiler_params=pltpu.CompilerParams(
            dimension_semantics=("arbitrary", "arbitrary")),
    )(t2, *([x] * _NSPLIT))
    return out[0, 0]


# speedup vs baseline: 1.9431x; 1.9431x over previous
import functools
import math

import jax
import jax.numpy as jnp
from jax.experimental import pallas as pl
from jax.experimental.pallas import tpu as pltpu

_SMOOTHING = 0.1
_CONF = 1.0 - _SMOOTHING
_RB = 512
_CB = 4096
_LANE = 128
_NSPLIT = 4
_SRB = _RB // _NSPLIT


def _loss_body(V, eps, c_row, gj, t_ref, *rest):
    x_refs = rest[:_NSPLIT]
    out_ref = rest[_NSPLIT]
    racc_ref = rest[_NSPLIT + 1]
    hacc_ref = rest[_NSPLIT + 2]
    i = pl.program_id(0)
    j = pl.program_id(1)
    nsl = _CB // _LANE

    @pl.when((i == 0) & (j == 0))
    def _init():
        out_ref[0, 0] = 0.0

    def fold_into(xr, q, t, first, mask_tail):
        block = xr[...]
        if mask_tail:
            cols = (gj - 1) * _CB + jax.lax.broadcasted_iota(
                jnp.int32, (1, _CB), 1)
            block = jnp.where(cols >= V, 0.0, block)
        for r in range(0, _SRB, 8):
            row = q * _SRB + r
            ts = t[row:row + 8, :]
            t_tile = (ts >> 7) - j * nsl
            acc = None
            hacc = None
            for k in range(nsl):
                sl = block[r:r + 8, k * _LANE:(k + 1) * _LANE]
                acc = sl if acc is None else acc + sl
                hsel = jnp.where(t_tile == k, sl, 0.0)
                hacc = hsel if hacc is None else hacc + hsel
            if first:
                racc_ref[row:row + 8, :] = acc
                hacc_ref[row:row + 8, :] = hacc
            else:
                racc_ref[row:row + 8, :] += acc
                hacc_ref[row:row + 8, :] += hacc

    @pl.when(j == 0)
    def _first():
        t = t_ref[...]
        s = 0.0
        for q, xr in enumerate(x_refs):
            fold_into(xr, q, t, True, gj == 1)
            x0 = xr[0:_SRB, 0:1]
            tq = t[q * _SRB:(q + 1) * _SRB, :]
            per_row = c_row + eps * x0
            s = s + jnp.sum(jnp.where(tq == 0, 0.0, per_row))
        out_ref[0, 0] += s

    if gj > 1:
        @pl.when((j > 0) & (j < gj - 1))
        def _mid():
            t = t_ref[...]
            for q, xr in enumerate(x_refs):
                fold_into(xr, q, t, False, False)

        @pl.when(j == gj - 1)
        def _tail():
            t = t_ref[...]
            for q, xr in enumerate(x_refs):
                fold_into(xr, q, t, False, True)

    @pl.when(j == gj - 1)
    def _last():
        t = t_ref[...]
        rowsum = jnp.sum(racc_ref[...], axis=1, keepdims=True)
        lane = jax.lax.broadcasted_iota(jnp.int32, (1, _LANE), 1)
        ghit = jnp.where((t & 127) == lane, hacc_ref[...], 0.0)
        g = jnp.sum(ghit, axis=1, keepdims=True)
        per_row = -eps * rowsum + (eps - _CONF) * g
        out_ref[0, 0] += jnp.sum(jnp.where(t == 0, 0.0, per_row))


def kernel(x, target):
    n, V = x.shape
    eps = _SMOOTHING / (V - 2)
    c_row = _SMOOTHING * math.log(eps) + _CONF * math.log(_CONF)
    t2 = target.astype(jnp.int32).reshape(n, 1)
    gi = n // _RB
    gj = pl.cdiv(V, _CB)

    out = pl.pallas_call(
        functools.partial(_loss_body, V, eps, c_row, gj),
        grid=(gi, gj),
        in_specs=[
            pl.BlockSpec((_RB, 1), lambda i, j: (i, 0)),
        ] + [
            pl.BlockSpec((_SRB, _CB),
                         lambda i, j, q=q: (_NSPLIT * i + q, j))
            for q in range(_NSPLIT)
        ],
        out_specs=pl.BlockSpec((1, 1), lambda i, j: (0, 0),
                               memory_space=pltpu.SMEM),
        out_shape=jax.ShapeDtypeStruct((1, 1), jnp.float32),
        scratch_shapes=[pltpu.VMEM((_RB, _LANE), jnp.float32),
                        pltpu.VMEM((_RB, _LANE), jnp.float32)],
        compiler_params=pltpu.CompilerParams(
            dimension_semantics=("arbitrary", "arbitrary")),
    )(t2, *([x] * _NSPLIT))
    return out[0, 0]
